# baseline (speedup 1.0000x reference)
import functools

import jax
import jax.numpy as jnp
from jax import lax
from jax.experimental import pallas as pl
from jax.experimental.pallas import tpu as pltpu
from jax.experimental.pallas import tpu_sc as plsc

NC, NS, L = 2, 16, 16
NW = NC * NS
CS = 16
NXB = 3
NPB = 2
LOOKAHEAD = 2


def kernel(x, pos_table):
    batch, seq, d = x.shape
    rows_per_w = seq // NW
    n_chunks = rows_per_w // CS
    ntasks = n_chunks * batch
    chunk = CS * d

    x_flat = x.reshape(-1)
    pos_flat = pos_table.reshape(-1)
    mesh = plsc.VectorSubcoreMesh(core_axis_name="c", subcore_axis_name="s")

    @functools.partial(
        pl.kernel,
        mesh=mesh,
        out_type=jax.ShapeDtypeStruct((batch * seq * d,), jnp.float32),
        scratch_types=(
            [pltpu.VMEM((chunk,), jnp.float32) for _ in range(NXB)]
            + [pltpu.VMEM((chunk,), jnp.float32) for _ in range(NPB)]
            + [pltpu.SemaphoreType.DMA for _ in range(NXB + NPB + NXB)]
        ),
    )
    def sc_body(x_hbm, pos_hbm, out_hbm, *scratch):
        xb = scratch[:NXB]
        pb = scratch[NXB:NXB + NPB]
        sems = scratch[NXB + NPB:]
        xsem = sems[:NXB]
        psem = sems[NXB:NXB + NPB]
        osem = sems[NXB + NPB:]

        wid = lax.axis_index("s") * NC + lax.axis_index("c")
        s0 = wid * rows_per_w

        in_cp = [None] * NXB
        out_cp = [None] * NXB
        p_cp = [None] * NPB

        def issue_in(t):
            slot = t % NXB
            if out_cp[slot] is not None:
                out_cp[slot].wait()
            c, b = t // batch, t % batch
            off = b * seq * d + (s0 + c * CS) * d
            in_cp[slot] = pltpu.async_copy(
                x_hbm.at[pl.ds(off, chunk)], xb[slot], xsem[slot])
            if b == 0:
                pslot = c % NPB
                p_cp[pslot] = pltpu.async_copy(
                    pos_hbm.at[pl.ds((s0 + c * CS) * d, chunk)],
                    pb[pslot], psem[pslot])

        for t in range(min(LOOKAHEAD, ntasks)):
            issue_in(t)
        for t in range(ntasks):
            if t + LOOKAHEAD < ntasks:
                issue_in(t + LOOKAHEAD)
            slot = t % NXB
            c, b = t // batch, t % batch
            pslot = c % NPB
            in_cp[slot].wait()
            if b == 0:
                p_cp[pslot].wait()

            xv, pv = xb[slot], pb[pslot]

            def body(i, _):
                sl = pl.ds(i * L, L)
                xv[sl] = xv[sl] + pv[sl]
                return 0

            lax.fori_loop(0, chunk // L, body, 0, unroll=8)

            off = b * seq * d + (s0 + c * CS) * d
            out_cp[slot] = pltpu.async_copy(
                xv, out_hbm.at[pl.ds(off, chunk)], osem[slot])
        for slot in range(NXB):
            if out_cp[slot] is not None:
                out_cp[slot].wait()

    out = sc_body(x_flat, pos_flat)
    return out.reshape(batch, seq, d)


# speedup vs baseline: 3.0093x; 3.0093x over previous
import functools

import jax
import jax.numpy as jnp
from jax import lax
from jax.experimental import pallas as pl
from jax.experimental.pallas import tpu as pltpu
from jax.experimental.pallas import tpu_sc as plsc

NC, NS, L = 2, 16, 16
NW = NC * NS
CS = 16
NXB = 3
NPB = 2
LOOKAHEAD = 2


def kernel(x, pos_table):
    batch, seq, d = x.shape
    rows_per_w = seq // NW
    n_chunks = rows_per_w // CS
    ntasks = n_chunks * batch

    mesh = plsc.VectorSubcoreMesh(core_axis_name="c", subcore_axis_name="s")

    @functools.partial(
        pl.kernel,
        mesh=mesh,
        out_type=jax.ShapeDtypeStruct((batch, seq, d), jnp.float32),
        scratch_types=(
            [pltpu.VMEM((CS, d), jnp.float32) for _ in range(NXB)]
            + [pltpu.VMEM((CS, d), jnp.float32) for _ in range(NPB)]
            + [pltpu.SemaphoreType.DMA for _ in range(NXB + NPB + NXB)]
        ),
    )
    def sc_body(x_hbm, pos_hbm, out_hbm, *scratch):
        xb = scratch[:NXB]
        pb = scratch[NXB:NXB + NPB]
        sems = scratch[NXB + NPB:]
        xsem = sems[:NXB]
        psem = sems[NXB:NXB + NPB]
        osem = sems[NXB + NPB:]

        wid = lax.axis_index("s") * NC + lax.axis_index("c")
        s0 = wid * rows_per_w

        in_cp = [None] * NXB
        out_cp = [None] * NXB
        p_cp = [None] * NPB

        def issue_in(t):
            slot = t % NXB
            if out_cp[slot] is not None:
                out_cp[slot].wait()
            c, b = t // batch, t % batch
            r0 = s0 + c * CS
            in_cp[slot] = pltpu.async_copy(
                x_hbm.at[b, pl.ds(r0, CS), :], xb[slot], xsem[slot])
            if b == 0:
                pslot = c % NPB
                p_cp[pslot] = pltpu.async_copy(
                    pos_hbm.at[pl.ds(r0, CS), :], pb[pslot], psem[pslot])

        for t in range(min(LOOKAHEAD, ntasks)):
            issue_in(t)
        for t in range(ntasks):
            if t + LOOKAHEAD < ntasks:
                issue_in(t + LOOKAHEAD)
            slot = t % NXB
            c, b = t // batch, t % batch
            pslot = c % NPB
            in_cp[slot].wait()
            if b == 0:
                p_cp[pslot].wait()

            xv, pv = xb[slot], pb[pslot]

            def body(j, _):
                sl = pl.ds(j * L, L)
                for r in range(CS):
                    xv[r, sl] = xv[r, sl] + pv[r, sl]
                return 0

            lax.fori_loop(0, d // L, body, 0)

            r0 = s0 + c * CS
            out_cp[slot] = pltpu.async_copy(
                xv, out_hbm.at[b, pl.ds(r0, CS), :], osem[slot])
        for slot in range(NXB):
            if out_cp[slot] is not None:
                out_cp[slot].wait()

    return sc_body(x, pos_table)


# speedup vs baseline: 3.6055x; 1.1981x over previous
import functools

import jax
import jax.numpy as jnp
from jax import lax
from jax.experimental import pallas as pl
from jax.experimental.pallas import tpu as pltpu
from jax.experimental.pallas import tpu_sc as plsc

NC, NS, L = 2, 16, 16
NW = NC * NS
CS = 16
NXB = 5
NPB = 2
LOOKAHEAD = 2


def kernel(x, pos_table):
    batch, seq, d = x.shape
    rows_per_w = seq // NW
    n_chunks = rows_per_w // CS
    ntasks = n_chunks * batch

    mesh = plsc.VectorSubcoreMesh(core_axis_name="c", subcore_axis_name="s")

    @functools.partial(
        pl.kernel,
        mesh=mesh,
        out_type=jax.ShapeDtypeStruct((batch, seq, d), jnp.float32),
        scratch_types=(
            [pltpu.VMEM((CS, d), jnp.float32) for _ in range(NXB)]
            + [pltpu.VMEM((CS, d), jnp.float32) for _ in range(NPB)]
            + [pltpu.SemaphoreType.DMA for _ in range(NXB + NPB + NXB)]
        ),
    )
    def sc_body(x_hbm, pos_hbm, out_hbm, *scratch):
        xb = scratch[:NXB]
        pb = scratch[NXB:NXB + NPB]
        sems = scratch[NXB + NPB:]
        xsem = sems[:NXB]
        psem = sems[NXB:NXB + NPB]
        osem = sems[NXB + NPB:]

        wid = lax.axis_index("s") * NC + lax.axis_index("c")
        s0 = wid * rows_per_w

        in_cp = [None] * NXB
        out_cp = [None] * NXB
        p_cp = [None] * NPB

        def issue_in(t):
            slot = t % NXB
            if out_cp[slot] is not None:
                out_cp[slot].wait()
            c, b = t // batch, t % batch
            r0 = s0 + c * CS
            in_cp[slot] = pltpu.async_copy(
                x_hbm.at[b, pl.ds(r0, CS), :], xb[slot], xsem[slot])
            if b == 0:
                pslot = c % NPB
                p_cp[pslot] = pltpu.async_copy(
                    pos_hbm.at[pl.ds(r0, CS), :], pb[pslot], psem[pslot])

        for t in range(min(LOOKAHEAD, ntasks)):
            issue_in(t)
        for t in range(ntasks):
            if t + LOOKAHEAD < ntasks:
                issue_in(t + LOOKAHEAD)
            slot = t % NXB
            c, b = t // batch, t % batch
            pslot = c % NPB
            in_cp[slot].wait()
            if b == 0:
                p_cp[pslot].wait()

            xv, pv = xb[slot], pb[pslot]

            def body(j, _):
                sl = pl.ds(j * L, L)
                for r in range(CS):
                    xv[r, sl] = xv[r, sl] + pv[r, sl]
                return 0

            lax.fori_loop(0, d // L, body, 0)

            r0 = s0 + c * CS
            out_cp[slot] = pltpu.async_copy(
                xv, out_hbm.at[b, pl.ds(r0, CS), :], osem[slot])
        for slot in range(NXB):
            if out_cp[slot] is not None:
                out_cp[slot].wait()

    return sc_body(x, pos_table)


# speedup vs baseline: 4.1426x; 1.1490x over previous
import functools

import jax
import jax.numpy as jnp
from jax import lax
from jax.experimental import pallas as pl
from jax.experimental.pallas import tpu as pltpu
from jax.experimental.pallas import tpu_sc as plsc

NC, NS, L = 2, 16, 16
NW = NC * NS
CS = 16
NXB = 5
NPB = 2
LOOKAHEAD = 2


def kernel(x, pos_table):
    batch, seq, d = x.shape
    rows_per_w = seq // NW
    n_chunks = rows_per_w // CS
    ntasks = n_chunks * batch

    mesh = plsc.VectorSubcoreMesh(core_axis_name="c", subcore_axis_name="s")

    @functools.partial(
        pl.kernel,
        mesh=mesh,
        out_type=jax.ShapeDtypeStruct((batch, seq, d), jnp.float32),
        scratch_types=(
            [pltpu.VMEM((CS, d), jnp.float32) for _ in range(NXB)]
            + [pltpu.VMEM((CS, d), jnp.float32) for _ in range(NPB)]
            + [pltpu.SemaphoreType.DMA for _ in range(NXB + NPB + NXB)]
        ),
    )
    def sc_body(x_hbm, pos_hbm, out_hbm, *scratch):
        xb = scratch[:NXB]
        pb = scratch[NXB:NXB + NPB]
        sems = scratch[NXB + NPB:]
        xsem = sems[:NXB]
        psem = sems[NXB:NXB + NPB]
        osem = sems[NXB + NPB:]

        wid = lax.axis_index("s") * NC + lax.axis_index("c")
        s0 = wid * rows_per_w

        in_cp = [None] * NXB
        out_cp = [None] * NXB
        p_cp = [None] * NPB

        def issue_in(t):
            slot = t % NXB
            if out_cp[slot] is not None:
                out_cp[slot].wait()
            c, b = t // batch, t % batch
            r0 = s0 + c * CS
            in_cp[slot] = pltpu.async_copy(
                x_hbm.at[b, pl.ds(r0, CS), :], xb[slot], xsem[slot])
            if b == 0:
                pslot = c % NPB
                p_cp[pslot] = pltpu.async_copy(
                    pos_hbm.at[pl.ds(r0, CS), :], pb[pslot], psem[pslot])

        for t in range(min(LOOKAHEAD, ntasks)):
            issue_in(t)
        for t in range(ntasks):
            if t + LOOKAHEAD < ntasks:
                issue_in(t + LOOKAHEAD)
            slot = t % NXB
            c, b = t // batch, t % batch
            pslot = c % NPB
            in_cp[slot].wait()
            if b == 0:
                p_cp[pslot].wait()

            xv, pv = xb[slot], pb[pslot]

            @plsc.parallel_loop(0, d // L)
            def _(j):
                sl = pl.ds(j * L, L)
                for r in range(CS):
                    xv[r, sl] = xv[r, sl] + pv[r, sl]

            r0 = s0 + c * CS
            out_cp[slot] = pltpu.async_copy(
                xv, out_hbm.at[b, pl.ds(r0, CS), :], osem[slot])
        for slot in range(NXB):
            if out_cp[slot] is not None:
                out_cp[slot].wait()

    return sc_body(x, pos_table)


# speedup vs baseline: 4.1491x; 1.0016x over previous
import functools

import jax
import jax.numpy as jnp
from jax import lax
from jax.experimental import pallas as pl
from jax.experimental.pallas import tpu as pltpu
from jax.experimental.pallas import tpu_sc as plsc

NC, NS, L = 2, 16, 16
NW = NC * NS
CS = 16
NXB = 5
NPB = 2
LOOKAHEAD = 3


def kernel(x, pos_table):
    batch, seq, d = x.shape
    rows_per_w = seq // NW
    n_chunks = rows_per_w // CS
    ntasks = n_chunks * batch

    mesh = plsc.VectorSubcoreMesh(core_axis_name="c", subcore_axis_name="s")

    @functools.partial(
        pl.kernel,
        mesh=mesh,
        out_type=jax.ShapeDtypeStruct((batch, seq, d), jnp.float32),
        scratch_types=(
            [pltpu.VMEM((CS, d), jnp.float32) for _ in range(NXB)]
            + [pltpu.VMEM((CS, d), jnp.float32) for _ in range(NPB)]
            + [pltpu.SemaphoreType.DMA for _ in range(NXB + NPB + NXB)]
        ),
    )
    def sc_body(x_hbm, pos_hbm, out_hbm, *scratch):
        xb = scratch[:NXB]
        pb = scratch[NXB:NXB + NPB]
        sems = scratch[NXB + NPB:]
        xsem = sems[:NXB]
        psem = sems[NXB:NXB + NPB]
        osem = sems[NXB + NPB:]

        wid = lax.axis_index("s") * NC + lax.axis_index("c")
        s0 = wid * rows_per_w

        in_cp = [None] * NXB
        out_cp = [None] * NXB
        p_cp = [None] * NPB

        def issue_in(t):
            slot = t % NXB
            if out_cp[slot] is not None:
                out_cp[slot].wait()
            c, b = t // batch, t % batch
            r0 = s0 + c * CS
            in_cp[slot] = pltpu.async_copy(
                x_hbm.at[b, pl.ds(r0, CS), :], xb[slot], xsem[slot])
            if b == 0:
                pslot = c % NPB
                p_cp[pslot] = pltpu.async_copy(
                    pos_hbm.at[pl.ds(r0, CS), :], pb[pslot], psem[pslot])

        for t in range(min(LOOKAHEAD, ntasks)):
            issue_in(t)
        for t in range(ntasks):
            if t + LOOKAHEAD < ntasks:
                issue_in(t + LOOKAHEAD)
            slot = t % NXB
            c, b = t // batch, t % batch
            pslot = c % NPB
            in_cp[slot].wait()
            if b == 0:
                p_cp[pslot].wait()

            xv, pv = xb[slot], pb[pslot]

            @plsc.parallel_loop(0, d // L)
            def _(j):
                sl = pl.ds(j * L, L)
                for r in range(CS):
                    xv[r, sl] = xv[r, sl] + pv[r, sl]

            r0 = s0 + c * CS
            out_cp[slot] = pltpu.async_copy(
                xv, out_hbm.at[b, pl.ds(r0, CS), :], osem[slot])
        for slot in range(NXB):
            if out_cp[slot] is not None:
                out_cp[slot].wait()

    return sc_body(x, pos_table)


# speedup vs baseline: 4.1532x; 1.0010x over previous
import functools

import jax
import jax.numpy as jnp
from jax import lax
from jax.experimental import pallas as pl
from jax.experimental.pallas import tpu as pltpu
from jax.experimental.pallas import tpu_sc as plsc

NC, NS, L = 2, 16, 16
NW = NC * NS
CS = 8
NXB = 12
NPB = 3
LOOKAHEAD = 4


def kernel(x, pos_table):
    batch, seq, d = x.shape
    rows_per_w = seq // NW
    n_chunks = rows_per_w // CS
    ntasks = n_chunks * batch

    mesh = plsc.VectorSubcoreMesh(core_axis_name="c", subcore_axis_name="s")

    @functools.partial(
        pl.kernel,
        mesh=mesh,
        out_type=jax.ShapeDtypeStruct((batch, seq, d), jnp.float32),
        scratch_types=(
            [pltpu.VMEM((CS, d), jnp.float32) for _ in range(NXB)]
            + [pltpu.VMEM((CS, d), jnp.float32) for _ in range(NPB)]
            + [pltpu.SemaphoreType.DMA for _ in range(NXB + NPB + NXB)]
        ),
    )
    def sc_body(x_hbm, pos_hbm, out_hbm, *scratch):
        xb = scratch[:NXB]
        pb = scratch[NXB:NXB + NPB]
        sems = scratch[NXB + NPB:]
        xsem = sems[:NXB]
        psem = sems[NXB:NXB + NPB]
        osem = sems[NXB + NPB:]

        wid = lax.axis_index("s") * NC + lax.axis_index("c")
        s0 = wid * rows_per_w

        in_cp = [None] * NXB
        out_cp = [None] * NXB
        p_cp = [None] * NPB

        def issue_in(t):
            slot = t % NXB
            if out_cp[slot] is not None:
                out_cp[slot].wait()
            c, b = t // batch, t % batch
            r0 = s0 + c * CS
            in_cp[slot] = pltpu.async_copy(
                x_hbm.at[b, pl.ds(r0, CS), :], xb[slot], xsem[slot])
            if b == 0:
                pslot = c % NPB
                p_cp[pslot] = pltpu.async_copy(
                    pos_hbm.at[pl.ds(r0, CS), :], pb[pslot], psem[pslot])

        for t in range(min(LOOKAHEAD, ntasks)):
            issue_in(t)
        for t in range(ntasks):
            if t + LOOKAHEAD < ntasks:
                issue_in(t + LOOKAHEAD)
            slot = t % NXB
            c, b = t // batch, t % batch
            in_cp[slot].wait()
            if b != batch - 1:
                continue
            pslot = c % NPB
            p_cp[pslot].wait()
            pv = pb[pslot]
            slots = [(c * batch + bb) % NXB for bb in range(batch)]
            bufs = [xb[s] for s in slots]

            @plsc.parallel_loop(0, d // L)
            def _(j):
                sl = pl.ds(j * L, L)
                for r in range(CS):
                    vp = pv[r, sl]
                    for bb in range(batch):
                        bufs[bb][r, sl] = bufs[bb][r, sl] + vp

            r0 = s0 + c * CS
            for bb in range(batch):
                out_cp[slots[bb]] = pltpu.async_copy(
                    bufs[bb], out_hbm.at[bb, pl.ds(r0, CS), :],
                    osem[slots[bb]])
        for slot in range(NXB):
            if out_cp[slot] is not None:
                out_cp[slot].wait()

    return sc_body(x, pos_table)


# speedup vs baseline: 4.4224x; 1.0648x over previous
import functools

import jax
import jax.numpy as jnp
from jax import lax
from jax.experimental import pallas as pl
from jax.experimental.pallas import tpu as pltpu
from jax.experimental.pallas import tpu_sc as plsc

NC, NS, L = 2, 16, 16
NW = NC * NS
CS = 8
NXB = 12
NPB = 3
LOOKAHEAD = 8


def kernel(x, pos_table):
    batch, seq, d = x.shape
    rows_per_w = seq // NW
    n_chunks = rows_per_w // CS
    ntasks = n_chunks * batch

    mesh = plsc.VectorSubcoreMesh(core_axis_name="c", subcore_axis_name="s")

    @functools.partial(
        pl.kernel,
        mesh=mesh,
        out_type=jax.ShapeDtypeStruct((batch, seq, d), jnp.float32),
        scratch_types=(
            [pltpu.VMEM((CS, d), jnp.float32) for _ in range(NXB)]
            + [pltpu.VMEM((CS, d), jnp.float32) for _ in range(NPB)]
            + [pltpu.SemaphoreType.DMA for _ in range(NXB + NPB + NXB)]
        ),
    )
    def sc_body(x_hbm, pos_hbm, out_hbm, *scratch):
        xb = scratch[:NXB]
        pb = scratch[NXB:NXB + NPB]
        sems = scratch[NXB + NPB:]
        xsem = sems[:NXB]
        psem = sems[NXB:NXB + NPB]
        osem = sems[NXB + NPB:]

        wid = lax.axis_index("s") * NC + lax.axis_index("c")
        s0 = wid * rows_per_w

        in_cp = [None] * NXB
        out_cp = [None] * NXB
        p_cp = [None] * NPB

        def issue_in(t):
            slot = t % NXB
            if out_cp[slot] is not None:
                out_cp[slot].wait()
            c, b = t // batch, t % batch
            r0 = s0 + c * CS
            in_cp[slot] = pltpu.async_copy(
                x_hbm.at[b, pl.ds(r0, CS), :], xb[slot], xsem[slot])
            if b == 0:
                pslot = c % NPB
                p_cp[pslot] = pltpu.async_copy(
                    pos_hbm.at[pl.ds(r0, CS), :], pb[pslot], psem[pslot])

        for t in range(min(LOOKAHEAD, ntasks)):
            issue_in(t)
        for t in range(ntasks):
            if t + LOOKAHEAD < ntasks:
                issue_in(t + LOOKAHEAD)
            slot = t % NXB
            c, b = t // batch, t % batch
            in_cp[slot].wait()
            if b != batch - 1:
                continue
            pslot = c % NPB
            p_cp[pslot].wait()
            pv = pb[pslot]
            slots = [(c * batch + bb) % NXB for bb in range(batch)]
            bufs = [xb[s] for s in slots]

            @plsc.parallel_loop(0, d // L)
            def _(j):
                sl = pl.ds(j * L, L)
                for r in range(CS):
                    vp = pv[r, sl]
                    for bb in range(batch):
                        bufs[bb][r, sl] = bufs[bb][r, sl] + vp

            r0 = s0 + c * CS
            for bb in range(batch):
                out_cp[slots[bb]] = pltpu.async_copy(
                    bufs[bb], out_hbm.at[bb, pl.ds(r0, CS), :],
                    osem[slots[bb]])
        for slot in range(NXB):
            if out_cp[slot] is not None:
                out_cp[slot].wait()

    return sc_body(x, pos_table)
